# Initial kernel scaffold; baseline (speedup 1.0000x reference)
import jax
import jax.numpy as jnp
from jax.experimental import pallas as pl


def kernel(x, table):
    raise NotImplementedError("write your pallas kernel here")


# speedup vs baseline: 1.3093x; 1.3093x over previous
import functools

import jax
import jax.numpy as jnp
from jax import lax
from jax.experimental import pallas as pl
from jax.experimental.pallas import tpu as pltpu
from jax.experimental.pallas import tpu_sc as plsc

_NUM_CORES = 2
_NUM_SUBCORES = 16
_NW = _NUM_CORES * _NUM_SUBCORES

_CHUNK = 128
_NBUF = 8


def _make_sc_gather(n_chunks: int, d: int):
    mesh = plsc.VectorSubcoreMesh(core_axis_name="c", subcore_axis_name="s")

    @functools.partial(
        pl.kernel,
        mesh=mesh,
        out_type=jax.ShapeDtypeStruct((_NW, n_chunks, _CHUNK, d), jnp.float32),
        compiler_params=pltpu.CompilerParams(use_tc_tiling_on_sc=False),
        scratch_types=(
            [pltpu.VMEM((n_chunks, _CHUNK), jnp.int32)]
            + [pltpu.VMEM((_CHUNK, d), jnp.float32) for _ in range(_NBUF)]
            + [pltpu.SemaphoreType.DMA for _ in range(2 * _NBUF)]
        ),
    )
    def gather_kernel(table_hbm, idx_hbm, out_hbm, idx_v, *bufs_and_sems):
        bufs = bufs_and_sems[:_NBUF]
        gsem = bufs_and_sems[_NBUF:2 * _NBUF]
        ssem = bufs_and_sems[2 * _NBUF:]
        wid = lax.axis_index("s") * _NUM_CORES + lax.axis_index("c")

        pltpu.sync_copy(idx_hbm.at[wid], idx_v)

        def gather_start(chunk, b):
            pltpu.make_async_copy(
                table_hbm.at[idx_v.at[chunk]], bufs[b], gsem[b]
            ).start()

        def gather_wait(chunk, b):
            pltpu.make_async_copy(
                table_hbm.at[idx_v.at[chunk]], bufs[b], gsem[b]
            ).wait()

        def store_start(chunk, b):
            pltpu.make_async_copy(
                bufs[b], out_hbm.at[wid, chunk], ssem[b]
            ).start()

        def store_wait(chunk, b):
            pltpu.make_async_copy(
                bufs[b], out_hbm.at[wid, chunk], ssem[b]
            ).wait()

        for b in range(_NBUF):
            gather_start(b, b)

        n_groups = n_chunks // _NBUF

        def group_body(g, carry):
            for b in range(_NBUF):
                c = g * _NBUF + b
                gather_wait(c, b)
                store_start(c, b)
            for b in range(_NBUF):
                c = g * _NBUF + b
                nc = c + _NBUF

                @pl.when(nc < n_chunks)
                def _():
                    store_wait(c, b)
                    gather_start(nc, b)

            return carry

        lax.fori_loop(0, n_groups, group_body, 0)

        for b in range(_NBUF):
            c = (n_groups - 1) * _NBUF + b
            store_wait(c, b)

    return gather_kernel


def kernel(x, table):
    batch, hist = x.shape
    vocab, d = table.shape
    total = batch * hist
    assert total % (_NW * _CHUNK) == 0
    n_chunks = total // (_NW * _CHUNK)
    assert n_chunks % _NBUF == 0

    idx = x.reshape(_NW, n_chunks, _CHUNK)
    out = _make_sc_gather(n_chunks, d)(table, idx)
    return out.reshape(batch, hist, d)
